# baseline (speedup 1.0000x reference)
import functools
import math

import jax
import jax.numpy as jnp
import numpy as np
from jax import lax
from jax.experimental import pallas as pl
from jax.experimental.pallas import tpu as pltpu
from jax.experimental.pallas import tpu_sc as plsc

VOCAB = 100000
D_MODEL = 768
MAX_POS = 2048
_SCALE = math.sqrt(float(D_MODEL))
_LANES = 16


def _positional_encoding_np(length, depth):
    depth_h = depth / 2
    positions = np.arange(length)[:, np.newaxis]
    depths = np.arange(depth_h)[np.newaxis, :] / depth_h
    angle_rates = 1 / 10000 ** depths
    angle_rads = positions * angle_rates
    return np.concatenate(
        [np.sin(angle_rads), np.cos(angle_rads)], axis=-1
    ).astype(np.float32)


@functools.partial(jax.jit, static_argnums=(3, 4))
def _run(xf, pos, table, n_rows, seq_len):
    info = plsc.get_sparse_core_info()
    nc, ns = info.num_cores, info.num_subcores
    nw = nc * ns
    b_per_w = n_rows // nw
    chunk = 64
    n_chunks = b_per_w // chunk

    mesh = plsc.VectorSubcoreMesh(core_axis_name="c", subcore_axis_name="s")

    @functools.partial(
        pl.kernel,
        mesh=mesh,
        out_type=jax.ShapeDtypeStruct((n_rows, D_MODEL), jnp.float32),
        scratch_types=[
            pltpu.VMEM((b_per_w,), jnp.int32),
            pltpu.VMEM((chunk, D_MODEL), jnp.float32),
            pltpu.VMEM((chunk, D_MODEL), jnp.float32),
            pltpu.SemaphoreType.DMA,
        ],
    )
    def body(x_hbm, pos_hbm, table_hbm, out_hbm, idx_v, g_v, p_v, sem):
        wid = lax.axis_index("s") * nc + lax.axis_index("c")
        base = wid * b_per_w
        t_base = lax.rem(base, seq_len)
        pltpu.sync_copy(x_hbm.at[pl.ds(base, b_per_w)], idx_v)
        cols16 = D_MODEL // _LANES
        for j in range(n_chunks):
            row0 = base + j * chunk
            pltpu.async_copy(
                table_hbm.at[idx_v.at[pl.ds(j * chunk, chunk)]], g_v, sem
            ).wait()
            pltpu.sync_copy(pos_hbm.at[pl.ds(t_base + j * chunk, chunk)], p_v)

            def col_body(c, r):
                sl = pl.ds(c * _LANES, _LANES)
                p_v[r, sl] = g_v[r, sl] * _SCALE + p_v[r, sl]
                return r

            def row_body(r, _):
                lax.fori_loop(0, cols16, col_body, r)
                return 0

            lax.fori_loop(0, chunk, row_body, 0)
            pltpu.sync_copy(p_v, out_hbm.at[pl.ds(row0, chunk)])

    return body(xf, pos, table)


def kernel(x, table):
    b, t = x.shape
    xf = x.reshape(b * t).astype(jnp.int32)
    pos = jnp.asarray(_positional_encoding_np(MAX_POS, D_MODEL))
    out = _run(xf, pos, table, b * t, t)
    return out.reshape(b, t, D_MODEL)


# speedup vs baseline: 2.0177x; 2.0177x over previous
import functools
import math

import jax
import jax.numpy as jnp
import numpy as np
from jax import lax
from jax.experimental import pallas as pl
from jax.experimental.pallas import tpu as pltpu
from jax.experimental.pallas import tpu_sc as plsc

VOCAB = 100000
D_MODEL = 768
MAX_POS = 2048
_SCALE = math.sqrt(float(D_MODEL))
_LANES = 16
_CHUNK = 32


def _positional_encoding_np(length, depth):
    depth_h = depth / 2
    positions = np.arange(length)[:, np.newaxis]
    depths = np.arange(depth_h)[np.newaxis, :] / depth_h
    angle_rates = 1 / 10000 ** depths
    angle_rads = positions * angle_rates
    return np.concatenate(
        [np.sin(angle_rads), np.cos(angle_rads)], axis=-1
    ).astype(np.float32)


@functools.partial(jax.jit, static_argnums=(3, 4))
def _run(xf, pos, table, n_rows, seq_len):
    info = plsc.get_sparse_core_info()
    nc, ns = info.num_cores, info.num_subcores
    nw = nc * ns
    b_per_w = n_rows // nw
    n_chunks = b_per_w // _CHUNK
    cols16 = D_MODEL // _LANES

    mesh = plsc.VectorSubcoreMesh(core_axis_name="c", subcore_axis_name="s")

    @functools.partial(
        pl.kernel,
        mesh=mesh,
        out_type=jax.ShapeDtypeStruct((n_rows, D_MODEL), jnp.float32),
        scratch_types=[
            pltpu.VMEM((b_per_w,), jnp.int32),
            pltpu.VMEM((_CHUNK, D_MODEL), jnp.float32),
            pltpu.VMEM((_CHUNK, D_MODEL), jnp.float32),
            pltpu.VMEM((_CHUNK, D_MODEL), jnp.float32),
            pltpu.VMEM((_CHUNK, D_MODEL), jnp.float32),
            pltpu.SemaphoreType.DMA,
            pltpu.SemaphoreType.DMA,
            pltpu.SemaphoreType.DMA,
            pltpu.SemaphoreType.DMA,
            pltpu.SemaphoreType.DMA,
            pltpu.SemaphoreType.DMA,
        ],
    )
    def body(x_hbm, pos_hbm, table_hbm, out_hbm,
             idx_v, g0, g1, p0, p1,
             gs0, gs1, ps0, ps1, os0, os1):
        g = (g0, g1)
        p = (p0, p1)
        gsem = (gs0, gs1)
        psem = (ps0, ps1)
        osem = (os0, os1)
        wid = lax.axis_index("s") * nc + lax.axis_index("c")
        base = wid * b_per_w
        t_base = lax.rem(base, seq_len)
        pltpu.sync_copy(x_hbm.at[pl.ds(base, b_per_w)], idx_v)

        def start(j):
            buf = j % 2
            gh = pltpu.async_copy(
                table_hbm.at[idx_v.at[pl.ds(j * _CHUNK, _CHUNK)]],
                g[buf], gsem[buf])
            ph = pltpu.async_copy(
                pos_hbm.at[pl.ds(t_base + j * _CHUNK, _CHUNK)],
                p[buf], psem[buf])
            return gh, ph

        store_h = [None, None]
        pend = start(0)
        for j in range(n_chunks):
            buf = j % 2
            if j + 1 < n_chunks:
                nbuf = (j + 1) % 2
                if store_h[nbuf] is not None:
                    store_h[nbuf].wait()
                    store_h[nbuf] = None
                nxt = start(j + 1)
            gh, ph = pend
            gh.wait()
            ph.wait()

            def row_body(r, _):
                for c in range(cols16):
                    sl = pl.ds(c * _LANES, _LANES)
                    plsc.addupdate(p[buf].at[r, sl], g[buf][r, sl] * _SCALE)
                return 0

            lax.fori_loop(0, _CHUNK, row_body, 0)
            store_h[buf] = pltpu.async_copy(
                p[buf], out_hbm.at[pl.ds(base + j * _CHUNK, _CHUNK)],
                osem[buf])
            if j + 1 < n_chunks:
                pend = nxt
        for h in store_h:
            if h is not None:
                h.wait()

    return body(xf, pos, table)


def kernel(x, table):
    b, t = x.shape
    xf = x.reshape(b * t).astype(jnp.int32)
    pos = jnp.asarray(_positional_encoding_np(MAX_POS, D_MODEL))
    out = _run(xf, pos, table, b * t, t)
    return out.reshape(b, t, D_MODEL)
